# baseline (speedup 1.0000x reference)
import functools

import jax
import jax.numpy as jnp
from jax import lax
from jax.experimental import pallas as pl
from jax.experimental.pallas import tpu as pltpu
from jax.experimental.pallas import tpu_sc as plsc

N = 10000
E = 320000
D_IN = 128
D_HID = 64
G = 16

NC = 2
NS = 16
NW = NC * NS
EPT = E // NW
C = 80
NCHUNK = EPT // C
NPAD = 10240
RPW = NPAD // NS
K = 5

_f32 = jnp.float32
_i32 = jnp.int32

_sc_params = pltpu.CompilerParams(
    needs_layout_passes=False, use_tc_tiling_on_sc=False
)


def _t1_body(h_ref, w_ref, o_ref):
    o_ref[...] = jnp.dot(h_ref[...], w_ref[...], preferred_element_type=_f32)


_t1 = pl.pallas_call(
    _t1_body,
    out_shape=jax.ShapeDtypeStruct((N, D_HID), _f32),
)


@functools.lru_cache(maxsize=None)
def _build_s1():
  mesh = plsc.VectorSubcoreMesh(
      core_axis_name="c", subcore_axis_name="s", num_cores=NC, num_subcores=NS
  )

  @functools.partial(
    pl.kernel,
    out_type=jax.ShapeDtypeStruct((NC, NPAD, D_HID), _f32),
    mesh=mesh,
    compiler_params=_sc_params,
    scratch_types=[
        pltpu.VMEM((NCHUNK, C), _i32),
        pltpu.VMEM((NCHUNK, C), _i32),
        pltpu.VMEM((NCHUNK, C), _f32),
        [pltpu.VMEM((C, D_HID), _f32) for _ in range(K)],
        [pltpu.VMEM((C, D_HID), _f32) for _ in range(K)],
        pltpu.VMEM_SHARED((NPAD, D_HID), _f32),
        [pltpu.SemaphoreType.DMA for _ in range(K)],
        [pltpu.SemaphoreType.DMA for _ in range(K)],
    ],
  )
  def _s1(hw, row3, col3, val3, zrows, out_ahw,
          rowa, cola, vala, rows, rowso, acc, gsem, ssem):
    c = lax.axis_index("c")
    s = lax.axis_index("s")
    wid = c * NS + s
    rslice = pl.ds(s * RPW, RPW)

    pltpu.sync_copy(row3.at[wid], rowa)
    pltpu.sync_copy(col3.at[wid], cola)
    pltpu.sync_copy(val3.at[wid], vala)
    pltpu.sync_copy(zrows.at[rslice], acc.at[rslice])
    plsc.subcore_barrier()

    def issue_gather(j, b):
        pltpu.async_copy(hw.at[cola.at[j]], rows[b], gsem[b])

    def drain_gather(b):
        pltpu.make_async_copy(hw.at[pl.ds(0, C)], rows[b], gsem[b]).wait()

    def drain_scatter(b):
        pltpu.make_async_copy(hw.at[pl.ds(0, C)], rowso[b], ssem[b]).wait()

    for b in range(K - 1):
        issue_gather(b, b)

    def step(tt, b):
        i = tt * K + b
        j = i + (K - 1)
        bj = (b + K - 1) % K

        if b == 0:
            issue_gather(j, bj)
        else:
            @pl.when(j < NCHUNK)
            def _():
                issue_gather(j, bj)

        drain_gather(b)
        @pl.when(tt >= 1)
        def _():
            drain_scatter(b)
        for g in range(C // 16):
            vals16 = vala[i, pl.ds(g * 16, 16)]
            for el in range(16):
                e = g * 16 + el
                vs = vals16[el]
                for kk in range(D_HID // 16):
                    sl = pl.ds(kk * 16, 16)
                    rowso[b][e, sl] = rows[b][e, sl] * vs
        pltpu.async_copy(rowso[b], acc.at[rowa.at[i]], ssem[b], add=True)

    def body(tt, carry):
        for b in range(K):
            step(tt, b)
        return carry

    lax.fori_loop(0, NCHUNK // K, body, 0)
    for b in range(K):
        drain_scatter(b)
    plsc.subcore_barrier()
    pltpu.sync_copy(acc.at[rslice], out_ahw.at[c].at[rslice])

  return _s1


def _t2_body(ahw_ref, b1_ref, wlt_ref, bl_ref, y_ref):
    ahw = ahw_ref[0, :N, :] + ahw_ref[1, :N, :]
    h1 = jnp.maximum(ahw + b1_ref[...], 0.0)
    h2 = jnp.dot(h1, wlt_ref[...], preferred_element_type=_f32) + bl_ref[...]
    h2 = jnp.maximum(h2, 0.0)
    m = jnp.max(h2, axis=1, keepdims=True)
    ex = jnp.exp(h2 - m)
    y = ex / jnp.sum(ex, axis=1, keepdims=True)
    y_ref[...] = y


_t2 = pl.pallas_call(
    _t2_body,
    out_shape=jax.ShapeDtypeStruct((N, G), _f32),
)


@functools.lru_cache(maxsize=None)
def _build_s2():
  mesh = plsc.VectorSubcoreMesh(
      core_axis_name="c", subcore_axis_name="s", num_cores=NC, num_subcores=NS
  )

  @functools.partial(
    pl.kernel,
    out_type=jax.ShapeDtypeStruct((NW, 2, 16), _f32),
    mesh=mesh,
    compiler_params=_sc_params,
    scratch_types=[
        pltpu.VMEM((NCHUNK, C), _i32),
        pltpu.VMEM((NCHUNK, C), _i32),
        pltpu.VMEM((NCHUNK, C), _f32),
        [pltpu.VMEM((C, G), _f32) for _ in range(K)],
        [pltpu.VMEM((C, G), _f32) for _ in range(K)],
        pltpu.VMEM((2, 16), _f32),
        [pltpu.SemaphoreType.DMA for _ in range(K)],
        [pltpu.SemaphoreType.DMA for _ in range(K)],
    ],
  )
  def _s2(ytab, row3, col3, val3, out,
          rowa, cola, vala, ybgr, y1mr, accb, ysem, zsem):
    c = lax.axis_index("c")
    s = lax.axis_index("s")
    wid = c * NS + s

    pltpu.sync_copy(row3.at[wid], rowa)
    pltpu.sync_copy(col3.at[wid], cola)
    pltpu.sync_copy(val3.at[wid], vala)

    def issue_gathers(j, b):
        pltpu.async_copy(ytab.at[rowa.at[j]], ybgr[b], ysem[b])
        pltpu.async_copy(ytab.at[cola.at[j]], y1mr[b], zsem[b])

    def drain_gathers(b):
        pltpu.make_async_copy(ytab.at[pl.ds(0, C)], ybgr[b], ysem[b]).wait()
        pltpu.make_async_copy(ytab.at[pl.ds(0, C)], y1mr[b], zsem[b]).wait()

    for b in range(K - 1):
        issue_gathers(b, b)

    def step(tt, b, carry):
        macc, gacc = carry
        i = tt * K + b
        j = i + (K - 1)
        bj = (b + K - 1) % K
        if b == 0:
            issue_gathers(j, bj)
        else:
            @pl.when(j < NCHUNK)
            def _():
                issue_gathers(j, bj)
        drain_gathers(b)
        for g in range(C // 16):
            vals16 = vala[i, pl.ds(g * 16, 16)]
            for el in range(16):
                e = g * 16 + el
                t = vals16[el] * ybgr[b][e]
                gacc = gacc + t
                macc = macc + t * (1.0 - y1mr[b][e])
        return macc, gacc

    def body(tt, carry):
        for b in range(K):
            carry = step(tt, b, carry)
        return carry

    macc, gacc = lax.fori_loop(
        0, NCHUNK // K, body,
        (jnp.zeros((16,), _f32), jnp.zeros((16,), _f32)),
    )
    accb[0] = macc
    accb[1] = gacc
    pltpu.sync_copy(accb, out.at[wid])

  return _s2


def kernel(H, edge_index, edge_values, W1, b1, Wl, bl):
    row3 = edge_index[0].reshape(NW, NCHUNK, C)
    col3 = edge_index[1].reshape(NW, NCHUNK, C)
    val3 = edge_values.reshape(NW, NCHUNK, C)
    hw = _t1(H, W1)
    zrows = jnp.zeros((NPAD, D_HID), _f32)
    ahwp = _build_s1()(hw, row3, col3, val3, zrows)
    y = _t2(ahwp, b1.reshape(1, D_HID), Wl.T, bl.reshape(1, G))
    parts = _build_s2()(y, row3, col3, val3)
    m = jnp.sum(parts[:, 0, :], axis=0)
    gamma = jnp.sum(parts[:, 1, :], axis=0)
    return jnp.sum(m / gamma).reshape(1)
